# baseline (speedup 1.0000x reference)
import jax
import jax.numpy as jnp
from jax.experimental import pallas as pl
from jax.experimental.pallas import tpu as pltpu

E = 64
H = 768
I = 3072


NSPLIT = 3
CHUNK = I // NSPLIT


def _ffn_block_kernel(x_ref, b1_ref, b2_ref, *w_and_o):
    w_refs = w_and_o[:-1]
    o_ref = w_and_o[-1]
    e = pl.program_id(0)
    per = x_ref.shape[0] // pl.num_programs(0)
    x = x_ref[pl.ds(e * per, per), :]
    o = b2_ref[pl.ds(e, 1), :]
    for q in range(NSPLIT):
        w1q = w_refs[2 * q]
        w2q = w_refs[2 * q + 1]
        h = jnp.dot(x, w1q[0], preferred_element_type=jnp.float32)
        h = h + b1_ref[pl.ds(e, 1), q * CHUNK:(q + 1) * CHUNK]
        o = o + jnp.dot(h, w2q[0], preferred_element_type=jnp.float32)
    o_ref[0] = o


def kernel(hidden_states, W1, b1, W2, b2):
    Bb, Ss, Hh = hidden_states.shape
    Ee = W1.shape[0]
    T = Bb * Ss
    x = hidden_states.reshape(T, Hh)

    w_specs = []
    w_args = []
    for q in range(NSPLIT):
        w_specs.append(
            pl.BlockSpec((1, Hh, CHUNK), lambda e, q=q: (e, 0, q)))
        w_args.append(W1)
        w_specs.append(
            pl.BlockSpec((1, CHUNK, Hh), lambda e, q=q: (e, q, 0)))
        w_args.append(W2)

    out = pl.pallas_call(
        _ffn_block_kernel,
        grid=(Ee,),
        in_specs=[
            pl.BlockSpec((T, Hh), lambda e: (0, 0)),
            pl.BlockSpec((Ee, I), lambda e: (0, 0)),
            pl.BlockSpec((Ee, Hh), lambda e: (0, 0)),
        ] + w_specs,
        out_specs=pl.BlockSpec((1, T // Ee, Hh), lambda e: (e, 0, 0)),
        out_shape=jax.ShapeDtypeStruct((Ee, T // Ee, Hh), jnp.float32),
        compiler_params=pltpu.CompilerParams(
            dimension_semantics=("arbitrary",),
        ),
    )(x, b1, b2, *w_args)
    return out.reshape(Bb, Ss, Hh)


# speedup vs baseline: 1.0032x; 1.0032x over previous
import jax
import jax.numpy as jnp
from jax.experimental import pallas as pl
from jax.experimental.pallas import tpu as pltpu

E = 64
H = 768
I = 3072


NSPLIT = 4
CHUNK = I // NSPLIT


def _ffn_block_kernel(x_ref, b1_ref, b2_ref, *w_and_o):
    w_refs = w_and_o[:-1]
    o_ref = w_and_o[-1]
    e = pl.program_id(0)
    per = x_ref.shape[0] // pl.num_programs(0)
    x = x_ref[pl.ds(e * per, per), :]
    o = b2_ref[pl.ds(e, 1), :]
    for q in range(NSPLIT):
        w1q = w_refs[2 * q]
        w2q = w_refs[2 * q + 1]
        h = jnp.dot(x, w1q[0], preferred_element_type=jnp.float32)
        h = h + b1_ref[pl.ds(e, 1), q * CHUNK:(q + 1) * CHUNK]
        o = o + jnp.dot(h, w2q[0], preferred_element_type=jnp.float32)
    o_ref[0] = o


def kernel(hidden_states, W1, b1, W2, b2):
    Bb, Ss, Hh = hidden_states.shape
    Ee = W1.shape[0]
    T = Bb * Ss
    x = hidden_states.reshape(T, Hh)

    w_specs = []
    w_args = []
    for q in range(NSPLIT):
        w_specs.append(
            pl.BlockSpec((1, Hh, CHUNK), lambda e, q=q: (e, 0, q)))
        w_args.append(W1)
        w_specs.append(
            pl.BlockSpec((1, CHUNK, Hh), lambda e, q=q: (e, q, 0)))
        w_args.append(W2)

    out = pl.pallas_call(
        _ffn_block_kernel,
        grid=(Ee,),
        in_specs=[
            pl.BlockSpec((T, Hh), lambda e: (0, 0)),
            pl.BlockSpec((Ee, I), lambda e: (0, 0)),
            pl.BlockSpec((Ee, Hh), lambda e: (0, 0)),
        ] + w_specs,
        out_specs=pl.BlockSpec((1, T // Ee, Hh), lambda e: (e, 0, 0)),
        out_shape=jax.ShapeDtypeStruct((Ee, T // Ee, Hh), jnp.float32),
        compiler_params=pltpu.CompilerParams(
            dimension_semantics=("parallel",),
        ),
    )(x, b1, b2, *w_args)
    return out.reshape(Bb, Ss, Hh)
